# baseline (speedup 1.0000x reference)
import functools

import jax
import jax.numpy as jnp
from jax import lax
from jax.experimental import pallas as pl
from jax.experimental.pallas import tpu as pltpu
from jax.experimental.pallas import tpu_sc as plsc

N_NODES = 10000
N_HE = 10000
N_INC = 320000
EPS = 1e-5
TOPO_W = 2.0

NCORE = 2
NSUB = 16
NW = NCORE * NSUB
CHUNK = 128
NCHUNK = -(-N_INC // (NW * CHUNK))
ZROWS = 632
ACC_ROWS = ZROWS * NSUB
NDUMP = ACC_ROWS - N_NODES
OROWS = ZROWS
CNT_W = 16
FW = 64
DEPTH = 3
CLAG = 8


def _make_seg_kernel(H):
    mesh = plsc.VectorSubcoreMesh(core_axis_name="c", subcore_axis_name="s")
    ngroups = -(-NCHUNK // DEPTH)

    @functools.partial(
        pl.kernel,
        mesh=mesh,
        out_type=jax.ShapeDtypeStruct((NCORE, H, ACC_ROWS, FW), jnp.float32),
        compiler_params=pltpu.CompilerParams(use_tc_tiling_on_sc=False),
        scratch_types=[
            pltpu.VMEM((NCHUNK, CHUNK), jnp.int32),
            pltpu.VMEM((NCHUNK, CHUNK), jnp.int32),
        ] + [pltpu.VMEM((CHUNK, FW), jnp.float32)] * DEPTH + [
            pltpu.VMEM_SHARED((ACC_ROWS, FW), jnp.float32),
            pltpu.VMEM_SHARED((ACC_ROWS, FW), jnp.float32),
        ] + [pltpu.SemaphoreType.DMA] * (2 * DEPTH),
    )
    def seg(table_hbm, src_hbm, dst_hbm, zeros_hbm, out_hbm,
            src_v, dst_v, *rest):
        rows = rest[:DEPTH]
        tbl = rest[DEPTH]
        acc = rest[DEPTH + 1]
        gsems = rest[DEPTH + 2:2 * DEPTH + 2]
        ssems = rest[2 * DEPTH + 2:3 * DEPTH + 2]
        c = lax.axis_index("c")
        s = lax.axis_index("s")
        wid = c * NSUB + s
        pltpu.sync_copy(src_hbm.at[wid], src_v)
        pltpu.sync_copy(dst_hbm.at[wid], dst_v)

        for h in range(H):
            pltpu.sync_copy(table_hbm.at[h, pl.ds(s * ZROWS, ZROWS)],
                            tbl.at[pl.ds(s * ZROWS, ZROWS)])
            pltpu.sync_copy(zeros_hbm.at[pl.ds(s * ZROWS, ZROWS)],
                            acc.at[pl.ds(s * ZROWS, ZROWS)])
            plsc.subcore_barrier()

            for b in range(DEPTH - 1):
                pltpu.async_copy(tbl.at[src_v.at[b]], rows[b], gsems[b])

            def body(g, carry):
                kb = g * DEPTH
                for b in range(DEPTH):
                    k = kb + b
                    bprev = (b - 1) % DEPTH

                    @pl.when(k < NCHUNK)
                    def _(k=k, b=b, bprev=bprev):
                        pltpu.make_async_copy(tbl.at[src_v.at[k]],
                                              rows[b], gsems[b]).wait()
                        pltpu.async_copy(rows[b], acc.at[dst_v.at[k]],
                                         ssems[b], add=True)

                        @pl.when(k + DEPTH - 1 < NCHUNK)
                        def _():
                            @pl.when(k > 0)
                            def _():
                                pltpu.make_async_copy(
                                    rows[bprev], acc.at[dst_v.at[0]],
                                    ssems[bprev]).wait()
                            pltpu.async_copy(tbl.at[src_v.at[k + DEPTH - 1]],
                                             rows[bprev], gsems[bprev])
                return carry

            lax.fori_loop(0, ngroups, body, 0)
            for b in range(DEPTH):
                pltpu.make_async_copy(rows[b], acc.at[dst_v.at[0]],
                                      ssems[b]).wait()
            plsc.subcore_barrier()
            pltpu.sync_copy(acc.at[pl.ds(s * OROWS, OROWS)],
                            out_hbm.at[c, h, pl.ds(s * OROWS, OROWS)])

    return seg


def _make_cnt_kernel():
    mesh = plsc.VectorSubcoreMesh(core_axis_name="c", subcore_axis_name="s")

    @functools.partial(
        pl.kernel,
        mesh=mesh,
        out_type=jax.ShapeDtypeStruct((NCORE, 2, ACC_ROWS, CNT_W), jnp.float32),
        compiler_params=pltpu.CompilerParams(use_tc_tiling_on_sc=False),
        scratch_types=[
            pltpu.VMEM((NCHUNK, CHUNK), jnp.int32),
            pltpu.VMEM((NCHUNK, CHUNK), jnp.int32),
            pltpu.VMEM((CHUNK, CNT_W), jnp.float32),
            pltpu.VMEM_SHARED((ACC_ROWS, CNT_W), jnp.float32),
            pltpu.VMEM_SHARED((ACC_ROWS, CNT_W), jnp.float32),
            pltpu.SemaphoreType.DMA,
            pltpu.SemaphoreType.DMA,
        ],
    )
    def cnt(nidx_hbm, eidx_hbm, ones_hbm, zeros_hbm, out_hbm,
            nidx_v, eidx_v, ones_v, accn, acce, sem_n, sem_e):
        c = lax.axis_index("c")
        s = lax.axis_index("s")
        wid = c * NSUB + s
        pltpu.sync_copy(zeros_hbm.at[pl.ds(s * ZROWS, ZROWS)],
                        accn.at[pl.ds(s * ZROWS, ZROWS)])
        pltpu.sync_copy(zeros_hbm.at[pl.ds(s * ZROWS, ZROWS)],
                        acce.at[pl.ds(s * ZROWS, ZROWS)])
        pltpu.sync_copy(ones_hbm, ones_v)
        pltpu.sync_copy(nidx_hbm.at[wid], nidx_v)
        pltpu.sync_copy(eidx_hbm.at[wid], eidx_v)
        plsc.subcore_barrier()

        def body(k, carry):
            @pl.when(k >= CLAG)
            def _():
                pltpu.make_async_copy(ones_v, accn.at[nidx_v.at[0]],
                                      sem_n).wait()
                pltpu.make_async_copy(ones_v, acce.at[eidx_v.at[0]],
                                      sem_e).wait()
            pltpu.async_copy(ones_v, accn.at[nidx_v.at[k]], sem_n, add=True)
            pltpu.async_copy(ones_v, acce.at[eidx_v.at[k]], sem_e, add=True)
            return carry

        lax.fori_loop(0, NCHUNK, body, 0)
        for _i in range(CLAG):
            pltpu.make_async_copy(ones_v, accn.at[nidx_v.at[0]], sem_n).wait()
            pltpu.make_async_copy(ones_v, acce.at[eidx_v.at[0]], sem_e).wait()
        plsc.subcore_barrier()
        pltpu.sync_copy(accn.at[pl.ds(s * OROWS, OROWS)],
                        out_hbm.at[c, 0, pl.ds(s * OROWS, OROWS)])
        pltpu.sync_copy(acce.at[pl.ds(s * OROWS, OROWS)],
                        out_hbm.at[c, 1, pl.ds(s * OROWS, OROWS)])

    return cnt


def _row_mask():
    ridx = lax.broadcasted_iota(jnp.int32, (ACC_ROWS, 1), 0)
    return ridx < N_NODES


def _bn(t, g, be):
    mask = _row_mask()
    tm = jnp.where(mask, t, 0.0)
    mu = jnp.sum(tm, axis=0, keepdims=True) / N_NODES
    dev = jnp.where(mask, t - mu, 0.0)
    var = jnp.sum(dev * dev, axis=0, keepdims=True) / N_NODES
    return g * (t - mu) / jnp.sqrt(var + EPS) + be


def _mm_body(x_ref, w_ref, o_ref):
    o_ref[0, 0:N_NODES] = jnp.dot(x_ref[...], w_ref[...],
                                  preferred_element_type=jnp.float32)
    o_ref[0, N_NODES:ACC_ROWS] = jnp.zeros((NDUMP, FW), jnp.float32)


def _tc_mm(x, w):
    return pl.pallas_call(
        _mm_body,
        out_shape=jax.ShapeDtypeStruct((1, ACC_ROWS, w.shape[1]), jnp.float32),
    )(x, w)


def _scale_body(p_ref, cnt_ref, o_ref):
    b = (cnt_ref[0, 1] + cnt_ref[1, 1])[:, 0:1]
    binv = jnp.where(b > 0, 1.0 / b, 0.0)[None]
    o_ref[...] = binv * (p_ref[0] + p_ref[1])


def _tc_scale(p, cnt):
    return pl.pallas_call(
        _scale_body,
        out_shape=jax.ShapeDtypeStruct(p.shape[1:], jnp.float32),
    )(p, cnt)


def _dinv_comb(q_ref, cnt_ref):
    d = (cnt_ref[0, 0] + cnt_ref[1, 0])[:, 0:1]
    dinv = jnp.where(d > 0, 1.0 / d, 0.0)[None]
    qs = dinv * (q_ref[0] + q_ref[1])
    if qs.shape[0] == 1:
        return qs[0]
    return jnp.concatenate([qs[0], qs[1]], axis=1)


def _post_body(q_ref, cnt_ref, b_ref, g_ref, be_ref, w_ref, o_ref):
    t = _dinv_comb(q_ref, cnt_ref) + b_ref[...]
    h = jnp.maximum(_bn(t, g_ref[...], be_ref[...]), 0.0)
    r = jnp.dot(h, w_ref[...], preferred_element_type=jnp.float32)
    for hh in range(o_ref.shape[0]):
        o_ref[hh] = r[:, hh * FW:(hh + 1) * FW]


def _tc_post(q, cnt, b, g, be, w):
    hout = w.shape[1] // FW
    return pl.pallas_call(
        _post_body,
        out_shape=jax.ShapeDtypeStruct((hout, ACC_ROWS, FW), jnp.float32),
    )(q, cnt, b.reshape(1, -1), g.reshape(1, -1), be.reshape(1, -1), w)


def _head_body(q_ref, cnt_ref, b_ref, g_ref, be_ref, bt_ref,
               wf1_ref, bf1_ref, wf2_ref, bf2_ref, o_ref):
    t = _dinv_comb(q_ref, cnt_ref) + b_ref[...]
    h = _bn(t, g_ref[...], be_ref[...])
    te2 = jnp.maximum(bt_ref[...], 0.0) * TOPO_W
    row = jnp.dot(te2, wf1_ref[64:128, :],
                  preferred_element_type=jnp.float32)
    o = jnp.dot(h, wf1_ref[0:64, :],
                preferred_element_type=jnp.float32) + row + bf1_ref[...]
    o = jnp.maximum(o, 0.0)
    lg = jnp.dot(o, wf2_ref[...], preferred_element_type=jnp.float32)
    lg = lg + bf2_ref[...]
    m = jnp.max(lg, axis=1, keepdims=True)
    z = lg - m
    lse = jnp.log(jnp.sum(jnp.exp(z), axis=1, keepdims=True))
    o_ref[...] = (z - lse)[0:N_NODES]


def _tc_head(q, cnt, b, g, be, bt, wf1, bf1, wf2, bf2):
    return pl.pallas_call(
        _head_body,
        out_shape=jax.ShapeDtypeStruct((N_NODES, wf2.shape[1]), jnp.float32),
    )(q, cnt, b.reshape(1, -1), g.reshape(1, -1), be.reshape(1, -1),
      bt.reshape(1, -1), wf1, bf1.reshape(1, -1), wf2, bf2.reshape(1, -1))


def kernel(x, edge_index, W1, b1, g1, be1, W2, b2, g2, be2, W3, b3, g3, be3,
           Wt, bt, Wf1, bf1, Wf2, bf2):
    node = edge_index[0].astype(jnp.int32)
    he = edge_index[1].astype(jnp.int32)

    npad = NW * CHUNK * NCHUNK - N_INC
    padvals = N_NODES + jnp.arange(npad, dtype=jnp.int32) % NDUMP

    def layout(idx):
        return jnp.concatenate([idx, padvals]).reshape(NW, NCHUNK, CHUNK)

    node_l = layout(node)
    he_l = layout(he)

    z64 = jnp.zeros((ACC_ROWS, FW), jnp.float32)
    zc = jnp.zeros((ACC_ROWS, CNT_W), jnp.float32)
    ones = jnp.ones((CHUNK, CNT_W), jnp.float32)

    seg1 = _make_seg_kernel(1)
    seg2 = _make_seg_kernel(2)
    cntk = _make_cnt_kernel()

    cnt = cntk(node_l, he_l, ones, zc)

    xw = _tc_mm(x, W1)
    p = seg1(xw, node_l, he_l, z64)
    t = _tc_scale(p, cnt)
    q = seg1(t, he_l, node_l, z64)
    xw = _tc_post(q, cnt, b1, g1, be1, W2)

    p = seg2(xw, node_l, he_l, z64)
    t = _tc_scale(p, cnt)
    q = seg2(t, he_l, node_l, z64)
    xw = _tc_post(q, cnt, b2, g2, be2, W3)

    p = seg1(xw, node_l, he_l, z64)
    t = _tc_scale(p, cnt)
    q = seg1(t, he_l, node_l, z64)

    return _tc_head(q, cnt, b3, g3, be3, bt, Wf1, bf1, Wf2, bf2)


# speedup vs baseline: 1.0014x; 1.0014x over previous
import functools

import jax
import jax.numpy as jnp
from jax import lax
from jax.experimental import pallas as pl
from jax.experimental.pallas import tpu as pltpu
from jax.experimental.pallas import tpu_sc as plsc

N_NODES = 10000
N_HE = 10000
N_INC = 320000
EPS = 1e-5
TOPO_W = 2.0

NCORE = 2
NSUB = 16
NW = NCORE * NSUB
CHUNK = 128
NCHUNK = -(-N_INC // (NW * CHUNK))
ZROWS = 632
ACC_ROWS = ZROWS * NSUB
NDUMP = ACC_ROWS - N_NODES
OROWS = ZROWS
CNT_W = 16
FW = 64
DEPTH = 3
CLAG = 8


def _make_seg_kernel(H):
    mesh = plsc.VectorSubcoreMesh(core_axis_name="c", subcore_axis_name="s")
    ngroups = -(-NCHUNK // DEPTH)

    @functools.partial(
        pl.kernel,
        mesh=mesh,
        out_type=jax.ShapeDtypeStruct((NCORE, H, ACC_ROWS, FW), jnp.float32),
        compiler_params=pltpu.CompilerParams(use_tc_tiling_on_sc=False),
        scratch_types=[
            pltpu.VMEM((NCHUNK, CHUNK), jnp.int32),
            pltpu.VMEM((NCHUNK, CHUNK), jnp.int32),
        ] + [pltpu.VMEM((CHUNK, FW), jnp.float32)] * DEPTH + [
            pltpu.VMEM_SHARED((ACC_ROWS, FW), jnp.float32),
            pltpu.VMEM_SHARED((ACC_ROWS, FW), jnp.float32),
        ] + [pltpu.SemaphoreType.DMA] * (2 * DEPTH),
    )
    def seg(table_hbm, src_hbm, dst_hbm, zeros_hbm, out_hbm,
            src_v, dst_v, *rest):
        rows = rest[:DEPTH]
        tbl = rest[DEPTH]
        acc = rest[DEPTH + 1]
        gsems = rest[DEPTH + 2:2 * DEPTH + 2]
        ssems = rest[2 * DEPTH + 2:3 * DEPTH + 2]
        c = lax.axis_index("c")
        s = lax.axis_index("s")
        wid = c * NSUB + s
        pltpu.sync_copy(src_hbm.at[wid], src_v)
        pltpu.sync_copy(dst_hbm.at[wid], dst_v)

        for h in range(H):
            pltpu.sync_copy(table_hbm.at[h, pl.ds(s * ZROWS, ZROWS)],
                            tbl.at[pl.ds(s * ZROWS, ZROWS)])
            pltpu.sync_copy(zeros_hbm.at[pl.ds(s * ZROWS, ZROWS)],
                            acc.at[pl.ds(s * ZROWS, ZROWS)])
            plsc.subcore_barrier()

            for b in range(DEPTH - 1):
                pltpu.async_copy(tbl.at[src_v.at[b]], rows[b], gsems[b])

            def body(g, carry):
                kb = g * DEPTH
                for b in range(DEPTH):
                    k = kb + b
                    bprev = (b - 1) % DEPTH

                    @pl.when(k < NCHUNK)
                    def _(k=k, b=b, bprev=bprev):
                        pltpu.make_async_copy(tbl.at[src_v.at[k]],
                                              rows[b], gsems[b]).wait()
                        pltpu.async_copy(rows[b], acc.at[dst_v.at[k]],
                                         ssems[b], add=True)

                        @pl.when(k + DEPTH - 1 < NCHUNK)
                        def _():
                            @pl.when(k > 0)
                            def _():
                                pltpu.make_async_copy(
                                    rows[bprev], acc.at[dst_v.at[0]],
                                    ssems[bprev]).wait()
                            pltpu.async_copy(tbl.at[src_v.at[k + DEPTH - 1]],
                                             rows[bprev], gsems[bprev])
                return carry

            lax.fori_loop(0, ngroups, body, 0)
            for b in range(DEPTH):
                pltpu.make_async_copy(rows[b], acc.at[dst_v.at[0]],
                                      ssems[b]).wait()
            plsc.subcore_barrier()
            pltpu.sync_copy(acc.at[pl.ds(s * OROWS, OROWS)],
                            out_hbm.at[c, h, pl.ds(s * OROWS, OROWS)])

    return seg


def _make_cnt_kernel():
    mesh = plsc.VectorSubcoreMesh(core_axis_name="c", subcore_axis_name="s")

    @functools.partial(
        pl.kernel,
        mesh=mesh,
        out_type=jax.ShapeDtypeStruct((NCORE, 2, ACC_ROWS, CNT_W), jnp.float32),
        compiler_params=pltpu.CompilerParams(use_tc_tiling_on_sc=False),
        scratch_types=[
            pltpu.VMEM((NCHUNK, CHUNK), jnp.int32),
            pltpu.VMEM((NCHUNK, CHUNK), jnp.int32),
            pltpu.VMEM((CHUNK, CNT_W), jnp.float32),
            pltpu.VMEM_SHARED((ACC_ROWS, CNT_W), jnp.float32),
            pltpu.VMEM_SHARED((ACC_ROWS, CNT_W), jnp.float32),
            pltpu.SemaphoreType.DMA,
            pltpu.SemaphoreType.DMA,
        ],
    )
    def cnt(nidx_hbm, eidx_hbm, ones_hbm, zeros_hbm, out_hbm,
            nidx_v, eidx_v, ones_v, accn, acce, sem_n, sem_e):
        c = lax.axis_index("c")
        s = lax.axis_index("s")
        wid = c * NSUB + s
        pltpu.sync_copy(zeros_hbm.at[pl.ds(s * ZROWS, ZROWS)],
                        accn.at[pl.ds(s * ZROWS, ZROWS)])
        pltpu.sync_copy(zeros_hbm.at[pl.ds(s * ZROWS, ZROWS)],
                        acce.at[pl.ds(s * ZROWS, ZROWS)])
        pltpu.sync_copy(ones_hbm, ones_v)
        pltpu.sync_copy(nidx_hbm.at[wid], nidx_v)
        pltpu.sync_copy(eidx_hbm.at[wid], eidx_v)
        plsc.subcore_barrier()

        def body(k, carry):
            @pl.when(k >= CLAG)
            def _():
                pltpu.make_async_copy(ones_v, accn.at[nidx_v.at[0]],
                                      sem_n).wait()
                pltpu.make_async_copy(ones_v, acce.at[eidx_v.at[0]],
                                      sem_e).wait()
            pltpu.async_copy(ones_v, accn.at[nidx_v.at[k]], sem_n, add=True)
            pltpu.async_copy(ones_v, acce.at[eidx_v.at[k]], sem_e, add=True)
            return carry

        lax.fori_loop(0, NCHUNK, body, 0)
        for _i in range(CLAG):
            pltpu.make_async_copy(ones_v, accn.at[nidx_v.at[0]], sem_n).wait()
            pltpu.make_async_copy(ones_v, acce.at[eidx_v.at[0]], sem_e).wait()
        plsc.subcore_barrier()
        pltpu.sync_copy(accn.at[pl.ds(s * OROWS, OROWS)],
                        out_hbm.at[c, 0, pl.ds(s * OROWS, OROWS)])
        pltpu.sync_copy(acce.at[pl.ds(s * OROWS, OROWS)],
                        out_hbm.at[c, 1, pl.ds(s * OROWS, OROWS)])

    return cnt


def _row_mask():
    ridx = lax.broadcasted_iota(jnp.int32, (ACC_ROWS, 1), 0)
    return ridx < N_NODES


def _bn(t, g, be):
    mask = _row_mask()
    tm = jnp.where(mask, t, 0.0)
    mu = jnp.sum(tm, axis=0, keepdims=True) / N_NODES
    dev = jnp.where(mask, t - mu, 0.0)
    var = jnp.sum(dev * dev, axis=0, keepdims=True) / N_NODES
    return g * (t - mu) / jnp.sqrt(var + EPS) + be


def _mm_body(x_ref, w_ref, o_ref):
    o_ref[0, 0:N_NODES] = jnp.dot(x_ref[...], w_ref[...],
                                  preferred_element_type=jnp.float32)
    o_ref[0, N_NODES:ACC_ROWS] = jnp.zeros((NDUMP, FW), jnp.float32)


def _tc_mm(x, w):
    return pl.pallas_call(
        _mm_body,
        out_shape=jax.ShapeDtypeStruct((1, ACC_ROWS, w.shape[1]), jnp.float32),
    )(x, w)


def _scale_body(p_ref, cnt_ref, o_ref):
    b = (cnt_ref[0, 1] + cnt_ref[1, 1])[:, 0:1]
    binv = jnp.where(b > 0, 1.0 / b, 0.0)[None]
    o_ref[...] = binv * (p_ref[0] + p_ref[1])


def _tc_scale(p, cnt):
    return pl.pallas_call(
        _scale_body,
        out_shape=jax.ShapeDtypeStruct(p.shape[1:], jnp.float32),
    )(p, cnt)


def _dinv_comb(q_ref, cnt_ref):
    d = (cnt_ref[0, 0] + cnt_ref[1, 0])[:, 0:1]
    dinv = jnp.where(d > 0, 1.0 / d, 0.0)[None]
    qs = dinv * (q_ref[0] + q_ref[1])
    if qs.shape[0] == 1:
        return qs[0]
    return jnp.concatenate([qs[0], qs[1]], axis=1)


def _post_body(q_ref, cnt_ref, b_ref, g_ref, be_ref, w_ref, o_ref):
    t = _dinv_comb(q_ref, cnt_ref) + b_ref[...]
    h = jnp.maximum(_bn(t, g_ref[...], be_ref[...]), 0.0)
    r = jnp.dot(h, w_ref[...], preferred_element_type=jnp.float32)
    for hh in range(o_ref.shape[0]):
        o_ref[hh] = r[:, hh * FW:(hh + 1) * FW]


def _tc_post(q, cnt, b, g, be, w):
    hout = w.shape[1] // FW
    return pl.pallas_call(
        _post_body,
        out_shape=jax.ShapeDtypeStruct((hout, ACC_ROWS, FW), jnp.float32),
    )(q, cnt, b.reshape(1, -1), g.reshape(1, -1), be.reshape(1, -1), w)


def _head_body(q_ref, cnt_ref, b_ref, g_ref, be_ref, bt_ref,
               wf1_ref, bf1_ref, wf2_ref, bf2_ref, o_ref):
    t = _dinv_comb(q_ref, cnt_ref) + b_ref[...]
    h = _bn(t, g_ref[...], be_ref[...])
    te2 = jnp.maximum(bt_ref[...], 0.0) * TOPO_W
    row = jnp.dot(te2, wf1_ref[64:128, :],
                  preferred_element_type=jnp.float32)
    o = jnp.dot(h, wf1_ref[0:64, :],
                preferred_element_type=jnp.float32) + row + bf1_ref[...]
    o = jnp.maximum(o, 0.0)
    lg = jnp.dot(o, wf2_ref[...], preferred_element_type=jnp.float32)
    lg = lg + bf2_ref[...]
    m = jnp.max(lg, axis=1, keepdims=True)
    z = lg - m
    lse = jnp.log(jnp.sum(jnp.exp(z), axis=1, keepdims=True))
    o_ref[...] = (z - lse)[0:N_NODES]


def _tc_head(q, cnt, b, g, be, bt, wf1, bf1, wf2, bf2):
    return pl.pallas_call(
        _head_body,
        out_shape=jax.ShapeDtypeStruct((N_NODES, wf2.shape[1]), jnp.float32),
    )(q, cnt, b.reshape(1, -1), g.reshape(1, -1), be.reshape(1, -1),
      bt.reshape(1, -1), wf1, bf1.reshape(1, -1), wf2, bf2.reshape(1, -1))


def kernel(x, edge_index, W1, b1, g1, be1, W2, b2, g2, be2, W3, b3, g3, be3,
           Wt, bt, Wf1, bf1, Wf2, bf2):
    node = edge_index[0].astype(jnp.int32)
    he = edge_index[1].astype(jnp.int32)

    npad = NW * CHUNK * NCHUNK - N_INC
    spread = jnp.arange(npad, dtype=jnp.int32)

    def layout(idx, padvals):
        return jnp.concatenate([idx, padvals]).reshape(NW, NCHUNK, CHUNK)

    src_pad = spread % N_NODES
    dst_pad = N_NODES + spread % NDUMP
    node_src = layout(node, src_pad)
    node_dst = layout(node, dst_pad)
    he_src = layout(he, src_pad)
    he_dst = layout(he, dst_pad)

    z64 = jnp.zeros((ACC_ROWS, FW), jnp.float32)
    zc = jnp.zeros((ACC_ROWS, CNT_W), jnp.float32)
    ones = jnp.ones((CHUNK, CNT_W), jnp.float32)

    seg1 = _make_seg_kernel(1)
    seg2 = _make_seg_kernel(2)
    cntk = _make_cnt_kernel()

    cnt = cntk(node_dst, he_dst, ones, zc)

    xw = _tc_mm(x, W1)
    p = seg1(xw, node_src, he_dst, z64)
    t = _tc_scale(p, cnt)
    q = seg1(t, he_src, node_dst, z64)
    xw = _tc_post(q, cnt, b1, g1, be1, W2)

    p = seg2(xw, node_src, he_dst, z64)
    t = _tc_scale(p, cnt)
    q = seg2(t, he_src, node_dst, z64)
    xw = _tc_post(q, cnt, b2, g2, be2, W3)

    p = seg1(xw, node_src, he_dst, z64)
    t = _tc_scale(p, cnt)
    q = seg1(t, he_src, node_dst, z64)

    return _tc_head(q, cnt, b3, g3, be3, bt, Wf1, bf1, Wf2, bf2)


# speedup vs baseline: 1.0024x; 1.0009x over previous
import functools

import jax
import jax.numpy as jnp
from jax import lax
from jax.experimental import pallas as pl
from jax.experimental.pallas import tpu as pltpu
from jax.experimental.pallas import tpu_sc as plsc

N_NODES = 10000
N_HE = 10000
N_INC = 320000
EPS = 1e-5
TOPO_W = 2.0

NCORE = 2
NSUB = 16
NW = NCORE * NSUB
CHUNK = 128
NCHUNK = -(-N_INC // (NW * CHUNK))
ZROWS = 632
ACC_ROWS = ZROWS * NSUB
NDUMP = ACC_ROWS - N_NODES
OROWS = ZROWS
CNT_W = 16
FW = 64
DEPTH = 3
CLAG = 8


def _make_seg_kernel(H):
    mesh = plsc.VectorSubcoreMesh(core_axis_name="c", subcore_axis_name="s")
    ngroups = -(-NCHUNK // DEPTH)

    @functools.partial(
        pl.kernel,
        mesh=mesh,
        out_type=jax.ShapeDtypeStruct((NCORE, H, ACC_ROWS, FW), jnp.float32),
        compiler_params=pltpu.CompilerParams(use_tc_tiling_on_sc=False),
        scratch_types=[
            pltpu.VMEM((NCHUNK, CHUNK), jnp.int32),
            pltpu.VMEM((NCHUNK, CHUNK), jnp.int32),
        ] + [pltpu.VMEM((CHUNK, FW), jnp.float32)] * DEPTH + [
            pltpu.VMEM_SHARED((ACC_ROWS, FW), jnp.float32),
            pltpu.VMEM_SHARED((ACC_ROWS, FW), jnp.float32),
        ] + [pltpu.SemaphoreType.DMA] * (2 * DEPTH),
    )
    def seg(table_hbm, src_hbm, dst_hbm, zeros_hbm, out_hbm,
            src_v, dst_v, *rest):
        rows = rest[:DEPTH]
        tbl = rest[DEPTH]
        acc = rest[DEPTH + 1]
        gsems = rest[DEPTH + 2:2 * DEPTH + 2]
        ssems = rest[2 * DEPTH + 2:3 * DEPTH + 2]
        c = lax.axis_index("c")
        s = lax.axis_index("s")
        wid = c * NSUB + s
        pltpu.sync_copy(src_hbm.at[wid], src_v)
        pltpu.sync_copy(dst_hbm.at[wid], dst_v)

        for h in range(H):
            pltpu.sync_copy(table_hbm.at[h, pl.ds(s * ZROWS, ZROWS)],
                            tbl.at[pl.ds(s * ZROWS, ZROWS)])
            pltpu.sync_copy(zeros_hbm.at[pl.ds(s * ZROWS, ZROWS)],
                            acc.at[pl.ds(s * ZROWS, ZROWS)])
            plsc.subcore_barrier()

            for b in range(DEPTH - 1):
                pltpu.async_copy(tbl.at[src_v.at[b]], rows[b], gsems[b])

            def body(g, carry):
                kb = g * DEPTH
                for b in range(DEPTH):
                    k = kb + b
                    bprev = (b - 1) % DEPTH

                    @pl.when(k < NCHUNK)
                    def _(k=k, b=b, bprev=bprev):
                        pltpu.make_async_copy(tbl.at[src_v.at[k]],
                                              rows[b], gsems[b]).wait()
                        pltpu.async_copy(rows[b], acc.at[dst_v.at[k]],
                                         ssems[b], add=True)

                        @pl.when(k + DEPTH - 1 < NCHUNK)
                        def _():
                            @pl.when(k > 0)
                            def _():
                                pltpu.make_async_copy(
                                    rows[bprev], acc.at[dst_v.at[0]],
                                    ssems[bprev]).wait()
                            pltpu.async_copy(tbl.at[src_v.at[k + DEPTH - 1]],
                                             rows[bprev], gsems[bprev])
                return carry

            lax.fori_loop(0, ngroups, body, 0)
            for b in range(DEPTH):
                pltpu.make_async_copy(rows[b], acc.at[dst_v.at[0]],
                                      ssems[b]).wait()
            plsc.subcore_barrier()
            pltpu.sync_copy(acc.at[pl.ds(s * OROWS, OROWS)],
                            out_hbm.at[c, h, pl.ds(s * OROWS, OROWS)])

    return seg


def _make_cnt_kernel():
    mesh = plsc.VectorSubcoreMesh(core_axis_name="c", subcore_axis_name="s")

    @functools.partial(
        pl.kernel,
        mesh=mesh,
        out_type=jax.ShapeDtypeStruct((NCORE, 2, ACC_ROWS, CNT_W), jnp.float32),
        compiler_params=pltpu.CompilerParams(use_tc_tiling_on_sc=False),
        scratch_types=[
            pltpu.VMEM((NCHUNK, CHUNK), jnp.int32),
            pltpu.VMEM((NCHUNK, CHUNK), jnp.int32),
            pltpu.VMEM((CHUNK, CNT_W), jnp.float32),
            pltpu.VMEM_SHARED((ACC_ROWS, CNT_W), jnp.float32),
            pltpu.VMEM_SHARED((ACC_ROWS, CNT_W), jnp.float32),
            pltpu.SemaphoreType.DMA,
            pltpu.SemaphoreType.DMA,
        ],
    )
    def cnt(nidx_hbm, eidx_hbm, ones_hbm, zeros_hbm, out_hbm,
            nidx_v, eidx_v, ones_v, accn, acce, sem_n, sem_e):
        c = lax.axis_index("c")
        s = lax.axis_index("s")
        wid = c * NSUB + s
        pltpu.sync_copy(zeros_hbm.at[pl.ds(s * ZROWS, ZROWS)],
                        accn.at[pl.ds(s * ZROWS, ZROWS)])
        pltpu.sync_copy(zeros_hbm.at[pl.ds(s * ZROWS, ZROWS)],
                        acce.at[pl.ds(s * ZROWS, ZROWS)])
        pltpu.sync_copy(ones_hbm, ones_v)
        pltpu.sync_copy(nidx_hbm.at[wid], nidx_v)
        pltpu.sync_copy(eidx_hbm.at[wid], eidx_v)
        plsc.subcore_barrier()

        def body(k, carry):
            @pl.when(k >= CLAG)
            def _():
                pltpu.make_async_copy(ones_v, accn.at[nidx_v.at[0]],
                                      sem_n).wait()
                pltpu.make_async_copy(ones_v, acce.at[eidx_v.at[0]],
                                      sem_e).wait()
            pltpu.async_copy(ones_v, accn.at[nidx_v.at[k]], sem_n, add=True)
            pltpu.async_copy(ones_v, acce.at[eidx_v.at[k]], sem_e, add=True)
            return carry

        lax.fori_loop(0, NCHUNK, body, 0)
        for _i in range(CLAG):
            pltpu.make_async_copy(ones_v, accn.at[nidx_v.at[0]], sem_n).wait()
            pltpu.make_async_copy(ones_v, acce.at[eidx_v.at[0]], sem_e).wait()
        plsc.subcore_barrier()
        pltpu.sync_copy(accn.at[pl.ds(s * OROWS, OROWS)],
                        out_hbm.at[c, 0, pl.ds(s * OROWS, OROWS)])
        pltpu.sync_copy(acce.at[pl.ds(s * OROWS, OROWS)],
                        out_hbm.at[c, 1, pl.ds(s * OROWS, OROWS)])

    return cnt


def _row_mask():
    ridx = lax.broadcasted_iota(jnp.int32, (ACC_ROWS, 1), 0)
    return ridx < N_NODES


def _bn(t, g, be):
    mask = _row_mask()
    tm = jnp.where(mask, t, 0.0)
    mu = jnp.sum(tm, axis=0, keepdims=True) / N_NODES
    dev = jnp.where(mask, t - mu, 0.0)
    var = jnp.sum(dev * dev, axis=0, keepdims=True) / N_NODES
    return g * (t - mu) / jnp.sqrt(var + EPS) + be


def _mm_body(x_ref, w_ref, o_ref):
    o_ref[0, 0:N_NODES] = jnp.dot(x_ref[...], w_ref[...],
                                  preferred_element_type=jnp.float32)
    o_ref[0, N_NODES:ACC_ROWS] = jnp.zeros((NDUMP, FW), jnp.float32)


def _tc_mm(x, w):
    return pl.pallas_call(
        _mm_body,
        out_shape=jax.ShapeDtypeStruct((1, ACC_ROWS, w.shape[1]), jnp.float32),
    )(x, w)


def _sinv_body(cnt_ref, dinv_ref, binv_ref):
    d = cnt_ref[0, 0] + cnt_ref[1, 0]
    dinv_ref[...] = jnp.where(d > 0, 1.0 / d, 0.0)
    b = cnt_ref[0, 1] + cnt_ref[1, 1]
    binv_ref[...] = jnp.where(b > 0, 1.0 / b, 0.0)


def _tc_sinv(cnt):
    return pl.pallas_call(
        _sinv_body,
        out_shape=[jax.ShapeDtypeStruct((ACC_ROWS, CNT_W), jnp.float32)] * 2,
    )(cnt)


def _scale_body(p_ref, binv_ref, o_ref):
    o_ref[...] = binv_ref[...][:, 0:1][None] * (p_ref[0] + p_ref[1])


def _tc_scale(p, binv):
    return pl.pallas_call(
        _scale_body,
        out_shape=jax.ShapeDtypeStruct(p.shape[1:], jnp.float32),
    )(p, binv)


def _dinv_comb(q_ref, dinv_ref):
    dinv = dinv_ref[...][:, 0:1][None]
    qs = dinv * (q_ref[0] + q_ref[1])
    if qs.shape[0] == 1:
        return qs[0]
    return jnp.concatenate([qs[0], qs[1]], axis=1)


def _post_body(q_ref, dinv_ref, b_ref, g_ref, be_ref, w_ref, o_ref):
    t = _dinv_comb(q_ref, dinv_ref) + b_ref[...]
    h = jnp.maximum(_bn(t, g_ref[...], be_ref[...]), 0.0)
    r = jnp.dot(h, w_ref[...], preferred_element_type=jnp.float32)
    for hh in range(o_ref.shape[0]):
        o_ref[hh] = r[:, hh * FW:(hh + 1) * FW]


def _tc_post(q, dinv, b, g, be, w):
    hout = w.shape[1] // FW
    return pl.pallas_call(
        _post_body,
        out_shape=jax.ShapeDtypeStruct((hout, ACC_ROWS, FW), jnp.float32),
    )(q, dinv, b.reshape(1, -1), g.reshape(1, -1), be.reshape(1, -1), w)


def _head_body(q_ref, dinv_ref, b_ref, g_ref, be_ref, bt_ref,
               wf1_ref, bf1_ref, wf2_ref, bf2_ref, o_ref):
    t = _dinv_comb(q_ref, dinv_ref) + b_ref[...]
    h = _bn(t, g_ref[...], be_ref[...])
    te2 = jnp.maximum(bt_ref[...], 0.0) * TOPO_W
    row = jnp.dot(te2, wf1_ref[64:128, :],
                  preferred_element_type=jnp.float32)
    o = jnp.dot(h, wf1_ref[0:64, :],
                preferred_element_type=jnp.float32) + row + bf1_ref[...]
    o = jnp.maximum(o, 0.0)
    lg = jnp.dot(o, wf2_ref[...], preferred_element_type=jnp.float32)
    lg = lg + bf2_ref[...]
    m = jnp.max(lg, axis=1, keepdims=True)
    z = lg - m
    lse = jnp.log(jnp.sum(jnp.exp(z), axis=1, keepdims=True))
    o_ref[...] = (z - lse)[0:N_NODES]


def _tc_head(q, dinv, b, g, be, bt, wf1, bf1, wf2, bf2):
    return pl.pallas_call(
        _head_body,
        out_shape=jax.ShapeDtypeStruct((N_NODES, wf2.shape[1]), jnp.float32),
    )(q, dinv, b.reshape(1, -1), g.reshape(1, -1), be.reshape(1, -1),
      bt.reshape(1, -1), wf1, bf1.reshape(1, -1), wf2, bf2.reshape(1, -1))


def kernel(x, edge_index, W1, b1, g1, be1, W2, b2, g2, be2, W3, b3, g3, be3,
           Wt, bt, Wf1, bf1, Wf2, bf2):
    node = edge_index[0].astype(jnp.int32)
    he = edge_index[1].astype(jnp.int32)

    npad = NW * CHUNK * NCHUNK - N_INC
    spread = jnp.arange(npad, dtype=jnp.int32)

    def layout(idx, padvals):
        return jnp.concatenate([idx, padvals]).reshape(NW, NCHUNK, CHUNK)

    src_pad = spread % N_NODES
    dst_pad = N_NODES + spread % NDUMP
    node_src = layout(node, src_pad)
    node_dst = layout(node, dst_pad)
    he_src = layout(he, src_pad)
    he_dst = layout(he, dst_pad)

    z64 = jnp.zeros((ACC_ROWS, FW), jnp.float32)
    zc = jnp.zeros((ACC_ROWS, CNT_W), jnp.float32)
    ones = jnp.ones((CHUNK, CNT_W), jnp.float32)

    seg1 = _make_seg_kernel(1)
    seg2 = _make_seg_kernel(2)
    cntk = _make_cnt_kernel()

    cnt = cntk(node_dst, he_dst, ones, zc)
    dinv, binv = _tc_sinv(cnt)

    xw = _tc_mm(x, W1)
    p = seg1(xw, node_src, he_dst, z64)
    t = _tc_scale(p, binv)
    q = seg1(t, he_src, node_dst, z64)
    xw = _tc_post(q, dinv, b1, g1, be1, W2)

    p = seg2(xw, node_src, he_dst, z64)
    t = _tc_scale(p, binv)
    q = seg2(t, he_src, node_dst, z64)
    xw = _tc_post(q, dinv, b2, g2, be2, W3)

    p = seg1(xw, node_src, he_dst, z64)
    t = _tc_scale(p, binv)
    q = seg1(t, he_src, node_dst, z64)

    return _tc_head(q, dinv, b3, g3, be3, bt, Wf1, bf1, Wf2, bf2)
